# baseline (speedup 1.0000x reference)
import jax
import jax.numpy as jnp
import numpy as np
from jax.experimental import pallas as pl
from jax.experimental.pallas import tpu as pltpu

_S3 = 3.0 ** -0.5
_S2 = 2.0 ** -0.5
_S6 = 6.0 ** -0.5
_CG2 = np.array([
    [_S3, 0, 0, 0, _S3, 0, 0, 0, _S3],
    [0, 0, 0, 0, 0, _S2, 0, -_S2, 0],
    [0, 0, -_S2, 0, 0, 0, _S2, 0, 0],
    [0, _S2, 0, -_S2, 0, 0, 0, 0, 0],
    [0, 0, _S2, 0, 0, 0, _S2, 0, 0],
    [0, 0, 0, 0, 0, _S2, 0, _S2, 0],
    [-_S6, 0, 0, 0, 2.0 * _S6, 0, 0, 0, -_S6],
    [0, _S2, 0, _S2, 0, 0, 0, 0, 0],
    [_S2, 0, 0, 0, 0, 0, 0, 0, -_S2],
], dtype=np.float32)


def _make_body(N, BLK, NB, GP):
    def body(x0_ref, x4_ref, x5_ref, x6_ref, x7_ref, x8_ref, ids_ref,
             W1t_ref, W2t_ref, b1_ref, b2_ref, w3_ref, w2_ref,
             inv_ref, b3_ref, M_ref, out_ref, acc_ref):
        i = pl.program_id(0)

        @pl.when(i == 0)
        def _():
            acc_ref[...] = jnp.zeros_like(acc_ref)

        base = i * BLK
        rows = jax.lax.broadcasted_iota(jnp.int32, (BLK, 1), 0) + base
        valid = rows < N

        x0 = jnp.where(valid, x0_ref[...], 0.0)
        h = jnp.dot(x0, W1t_ref[...], preferred_element_type=jnp.float32)
        h = jax.nn.silu(h + b1_ref[...])
        h = jnp.dot(h, W2t_ref[...], preferred_element_type=jnp.float32)
        h = jax.nn.silu(h + b2_ref[...])
        s = jnp.sum(h * w3_ref[...], axis=1, keepdims=True)

        w2r = w2_ref[...]
        parts = [s]
        for xr in (x4_ref, x5_ref, x6_ref, x7_ref, x8_ref):
            xm = jnp.where(valid, xr[...], 0.0)
            parts.append(jnp.sum(xm * w2r, axis=1, keepdims=True))
        parts.append(jnp.zeros((BLK, 2), jnp.float32))
        vals = jnp.concatenate(parts, axis=1)
        vals = jnp.where(valid, vals, 0.0)

        ids = ids_ref[0]
        gid = jax.lax.broadcasted_iota(jnp.int32, (GP, BLK), 0)
        oh = (gid == ids).astype(jnp.bfloat16)
        acc_ref[...] += jnp.dot(oh, vals.astype(jnp.bfloat16),
                                preferred_element_type=jnp.float32)

        @pl.when(i == NB - 1)
        def _():
            dec = acc_ref[...] * inv_ref[...] + b3_ref[...]
            out_ref[...] = jnp.dot(dec, M_ref[...],
                                   preferred_element_type=jnp.float32)
    return body


def kernel(node_embedding, batch, natoms, W1, b1, W2, b2, W3, b3, l2_w, l2_b):
    N, M9, C = node_embedding.shape
    H = W1.shape[0]
    G = natoms.shape[0]
    BLK = 512
    NB = -(-N // BLK)
    GP = -(-G // 128) * 128

    x2d = node_embedding.reshape(N, M9 * C)
    ids3d = jnp.concatenate(
        [batch, jnp.zeros((NB * BLK - N,), jnp.int32)]).reshape(NB, 1, BLK)

    W1t = W1.T
    W2t = W2.T
    b1r = b1.reshape(1, H)
    b2r = b2.reshape(1, H)
    w3r = W3.reshape(1, H)
    w2r = l2_w[2].reshape(1, C)

    inv = 1.0 / natoms.astype(jnp.float32)
    inv8 = jnp.broadcast_to(
        jnp.pad(inv, (0, GP - G), constant_values=1.0)[:, None], (GP, 8))
    b3v = jnp.zeros((1, 8), jnp.float32).at[0, 0].set(b3[0])

    Mfull = np.zeros((8, 128), np.float32)
    Mfull[0, :9] = _CG2[0]
    for j in range(5):
        Mfull[1 + j, :9] = _CG2[4 + j]
    Mfull = jnp.asarray(Mfull)

    x_specs = [
        pl.BlockSpec((BLK, C), (lambda i, m=m: (i, m)))
        for m in (0, 4, 5, 6, 7, 8)
    ]
    in_specs = x_specs + [
        pl.BlockSpec((1, 1, BLK), lambda i: (i, 0, 0)),
        pl.BlockSpec((C, H), lambda i: (0, 0)),
        pl.BlockSpec((H, H), lambda i: (0, 0)),
        pl.BlockSpec((1, H), lambda i: (0, 0)),
        pl.BlockSpec((1, H), lambda i: (0, 0)),
        pl.BlockSpec((1, H), lambda i: (0, 0)),
        pl.BlockSpec((1, C), lambda i: (0, 0)),
        pl.BlockSpec((GP, 8), lambda i: (0, 0)),
        pl.BlockSpec((1, 8), lambda i: (0, 0)),
        pl.BlockSpec((8, 128), lambda i: (0, 0)),
    ]

    out = pl.pallas_call(
        _make_body(N, BLK, NB, GP),
        grid=(NB,),
        in_specs=in_specs,
        out_specs=pl.BlockSpec((GP, 128), lambda i: (0, 0)),
        out_shape=jax.ShapeDtypeStruct((GP, 128), jnp.float32),
        scratch_shapes=[pltpu.VMEM((GP, 8), jnp.float32)],
    )(x2d, x2d, x2d, x2d, x2d, x2d, ids3d,
      W1t, W2t, b1r, b2r, w3r, w2r, inv8, b3v, Mfull)

    return out[:G, :9].reshape(G, 3, 3)


# speedup vs baseline: 1.3636x; 1.3636x over previous
import jax
import jax.numpy as jnp
import numpy as np
from jax.experimental import pallas as pl
from jax.experimental.pallas import tpu as pltpu

_S3 = 3.0 ** -0.5
_S2 = 2.0 ** -0.5
_S6 = 6.0 ** -0.5
_CG2 = np.array([
    [_S3, 0, 0, 0, _S3, 0, 0, 0, _S3],
    [0, 0, 0, 0, 0, _S2, 0, -_S2, 0],
    [0, 0, -_S2, 0, 0, 0, _S2, 0, 0],
    [0, _S2, 0, -_S2, 0, 0, 0, 0, 0],
    [0, 0, _S2, 0, 0, 0, _S2, 0, 0],
    [0, 0, 0, 0, 0, _S2, 0, _S2, 0],
    [-_S6, 0, 0, 0, 2.0 * _S6, 0, 0, 0, -_S6],
    [0, _S2, 0, _S2, 0, 0, 0, 0, 0],
    [_S2, 0, 0, 0, 0, 0, 0, 0, -_S2],
], dtype=np.float32)

_PLANES = (0, 4, 5, 6, 7, 8)


def _make_body(N, BLK, NB, GP):
    def body(x_hbm, ids_ref, W1t_ref, W2t_ref, b1_ref, b2_ref, w3_ref,
             w2_ref, inv_ref, b3_ref, M_ref, out_ref, xbuf, sem, acc_ref):
        i = pl.program_id(0)

        def base_of(idx):
            return jnp.minimum(idx * BLK, N - BLK)

        def dma(idx, slot, p):
            m = _PLANES[p]
            return pltpu.make_async_copy(
                x_hbm.at[pl.ds(base_of(idx), BLK), m, :],
                xbuf.at[slot, p],
                sem.at[slot, p])

        def start(idx, slot):
            for p in range(6):
                dma(idx, slot, p).start()

        @pl.when(i == 0)
        def _():
            acc_ref[...] = jnp.zeros_like(acc_ref)
            start(0, 0)

        slot = jax.lax.rem(i, 2)

        @pl.when(i + 1 < NB)
        def _():
            start(i + 1, jax.lax.rem(i + 1, 2))

        for p in range(6):
            dma(i, slot, p).wait()

        base = base_of(i)
        rows = jax.lax.broadcasted_iota(jnp.int32, (BLK, 1), 0) + base
        valid = rows >= i * BLK

        x0 = jnp.where(valid, xbuf[slot, 0], 0.0)
        h = jnp.dot(x0, W1t_ref[...], preferred_element_type=jnp.float32)
        h = jax.nn.silu(h + b1_ref[...])
        h = jnp.dot(h, W2t_ref[...], preferred_element_type=jnp.float32)
        h = jax.nn.silu(h + b2_ref[...])
        s = jnp.sum(h * w3_ref[...], axis=1, keepdims=True)

        w2r = w2_ref[...]
        parts = [s]
        for p in range(1, 6):
            xm = jnp.where(valid, xbuf[slot, p], 0.0)
            parts.append(jnp.sum(xm * w2r, axis=1, keepdims=True))
        parts.append(jnp.zeros((BLK, 2), jnp.float32))
        vals = jnp.concatenate(parts, axis=1)
        vals = jnp.where(valid, vals, 0.0)

        ids = ids_ref[0]
        gid = jax.lax.broadcasted_iota(jnp.int32, (GP, BLK), 0)
        oh = (gid == ids).astype(jnp.bfloat16)
        acc_ref[...] += jnp.dot(oh, vals.astype(jnp.bfloat16),
                                preferred_element_type=jnp.float32)

        @pl.when(i == NB - 1)
        def _():
            dec = acc_ref[...] * inv_ref[...] + b3_ref[...]
            out_ref[...] = jnp.dot(dec, M_ref[...],
                                   preferred_element_type=jnp.float32)
    return body


def kernel(node_embedding, batch, natoms, W1, b1, W2, b2, W3, b3, l2_w, l2_b):
    N, M9, C = node_embedding.shape
    H = W1.shape[0]
    G = natoms.shape[0]
    BLK = 512
    NB = -(-N // BLK)
    GP = -(-G // 128) * 128

    bases = np.minimum(np.arange(NB) * BLK, N - BLK)
    idx = bases[:, None] + np.arange(BLK)[None, :]
    ids3d = batch[jnp.asarray(idx)].reshape(NB, 1, BLK)

    W1t = W1.T
    W2t = W2.T
    b1r = b1.reshape(1, H)
    b2r = b2.reshape(1, H)
    w3r = W3.reshape(1, H)
    w2r = l2_w[2].reshape(1, C)

    inv = 1.0 / natoms.astype(jnp.float32)
    inv8 = jnp.broadcast_to(
        jnp.pad(inv, (0, GP - G), constant_values=1.0)[:, None], (GP, 8))
    b3v = jnp.zeros((1, 8), jnp.float32).at[0, 0].set(b3[0])

    Mfull = np.zeros((8, 128), np.float32)
    Mfull[0, :9] = _CG2[0]
    for j in range(5):
        Mfull[1 + j, :9] = _CG2[4 + j]
    Mfull = jnp.asarray(Mfull)

    in_specs = [
        pl.BlockSpec(memory_space=pltpu.MemorySpace.HBM),
        pl.BlockSpec((1, 1, BLK), lambda i: (i, 0, 0)),
        pl.BlockSpec((C, H), lambda i: (0, 0)),
        pl.BlockSpec((H, H), lambda i: (0, 0)),
        pl.BlockSpec((1, H), lambda i: (0, 0)),
        pl.BlockSpec((1, H), lambda i: (0, 0)),
        pl.BlockSpec((1, H), lambda i: (0, 0)),
        pl.BlockSpec((1, C), lambda i: (0, 0)),
        pl.BlockSpec((GP, 8), lambda i: (0, 0)),
        pl.BlockSpec((1, 8), lambda i: (0, 0)),
        pl.BlockSpec((8, 128), lambda i: (0, 0)),
    ]

    out = pl.pallas_call(
        _make_body(N, BLK, NB, GP),
        grid=(NB,),
        in_specs=in_specs,
        out_specs=pl.BlockSpec((GP, 128), lambda i: (0, 0)),
        out_shape=jax.ShapeDtypeStruct((GP, 128), jnp.float32),
        scratch_shapes=[
            pltpu.VMEM((2, 6, BLK, C), jnp.float32),
            pltpu.SemaphoreType.DMA((2, 6)),
            pltpu.VMEM((GP, 8), jnp.float32),
        ],
    )(node_embedding, ids3d, W1t, W2t, b1r, b2r, w3r, w2r, inv8, b3v, Mfull)

    return out[:G, :9].reshape(G, 3, 3)
